# baseline (speedup 1.0000x reference)
import functools

import jax
import jax.numpy as jnp
from jax import lax
from jax.experimental import pallas as pl
from jax.experimental.pallas import tpu as pltpu
from jax.experimental.pallas import tpu_sc as plsc

L = 16


def _build_sc_kernel(N, M, K):
    NW = 32
    cols_per_w = M // NW
    nchunk = 5
    chunk_rows = N // nchunk
    kv = cols_per_w // L
    K_pad = -(-K // L) * L

    mesh = plsc.VectorSubcoreMesh(core_axis_name="c", subcore_axis_name="s")

    @functools.partial(
        pl.kernel,
        mesh=mesh,
        compiler_params=pltpu.CompilerParams(needs_layout_passes=False),
        out_type=jax.ShapeDtypeStruct((N, M), jnp.float32),
        scratch_types=[
            pltpu.VMEM((2 * L,), jnp.float32),
            pltpu.VMEM((K,), jnp.int32),
            pltpu.VMEM((K, cols_per_w), jnp.float32),
            pltpu.VMEM((chunk_rows, cols_per_w), jnp.float32),
            pltpu.VMEM((chunk_rows, cols_per_w), jnp.float32),
            pltpu.VMEM((chunk_rows, cols_per_w), jnp.float32),
            pltpu.VMEM((chunk_rows, cols_per_w), jnp.float32),
            pltpu.SemaphoreType.DMA,
            pltpu.SemaphoreType.DMA,
            pltpu.SemaphoreType.DMA,
            pltpu.SemaphoreType.DMA,
            pltpu.SemaphoreType.DMA,
        ],
    )
    def sc_kernel(xt_hbm, ab_hbm, clss_hbm, out_hbm,
                  ab_v, clss_v, fix_v, in0, in1, out0, out1,
                  isem0, isem1, osem0, osem1, fsem):
        wid = lax.axis_index("s") * 2 + lax.axis_index("c")
        col0 = wid * cols_per_w

        ins = [in0, in1]
        outs = [out0, out1]
        isems = [isem0, isem1]
        osems = [osem0, osem1]

        in_copies = {}
        for c in range(min(2, nchunk)):
            in_copies[c] = pltpu.async_copy(
                xt_hbm.at[pl.ds(c * chunk_rows, chunk_rows),
                          pl.ds(col0, cols_per_w)],
                ins[c % 2], isems[c % 2])

        pltpu.sync_copy(ab_hbm, ab_v)
        pltpu.sync_copy(clss_hbm, clss_v.at[pl.ds(0, K)])

        ones = jnp.full((L,), 1.0, jnp.float32)

        @plsc.parallel_loop(0, N // L)
        def init_body(i):
            a_v[pl.ds(i * L, L)] = ones
            b_v[pl.ds(i * L, L)] = ones

        zero16 = jnp.zeros((L,), jnp.int32)
        alpha_vec = ab_v[pl.ds(0, L)]
        beta_vec = ab_v[pl.ds(L, L)]
        lane = lax.iota(jnp.int32, L)
        for k in range(K_pad // L):
            idx = clss_v[pl.ds(k * L, L)]
            mask = (lane + (k * L)) < K
            plsc.store_scatter(a_v, [idx], alpha_vec, mask)
            plsc.store_scatter(b_v, [idx], beta_vec, mask)

        def do_chunk(c, b):
            row0 = pl.multiple_of(c * chunk_rows, 8)
            pltpu.make_async_copy(
                xt_hbm.at[pl.ds(row0, chunk_rows),
                          pl.ds(col0, cols_per_w)],
                ins[b], isems[b]).wait()

            @pl.when(c >= 2)
            def _wait_out(b=b, c=c):
                prev0 = pl.multiple_of((c - 2) * chunk_rows, 8)
                pltpu.make_async_copy(
                    outs[b],
                    out_hbm.at[pl.ds(prev0, chunk_rows),
                               pl.ds(col0, cols_per_w)],
                    osems[b]).wait()

            @plsc.parallel_loop(0, chunk_rows)
            def row_body(i, b=b):
                for k in range(kv):
                    outs[b][i, pl.ds(k * L, L)] = (
                        ins[b][i, pl.ds(k * L, L)] + 1.0)

            pltpu.async_copy(
                outs[b],
                out_hbm.at[pl.ds(row0, chunk_rows),
                           pl.ds(col0, cols_per_w)],
                osems[b])

            @pl.when(c + 2 < nchunk)
            def _start_next(b=b, c=c):
                nxt0 = pl.multiple_of((c + 2) * chunk_rows, 8)
                pltpu.async_copy(
                    xt_hbm.at[pl.ds(nxt0, chunk_rows),
                              pl.ds(col0, cols_per_w)],
                    ins[b], isems[b])

        @pl.loop(0, nchunk - 1, step=2)
        def group_body(g):
            for b in (0, 1):
                do_chunk(g + b, b)

        do_chunk(nchunk - 1, (nchunk - 1) % 2)

        fix_gather.wait()

        @plsc.parallel_loop(0, K)
        def fix_row(i):
            for k in range(kv):
                fix_v[i, pl.ds(k * L, L)] = (
                    alpha_vec * fix_v[i, pl.ds(k * L, L)] + beta_vec)

        for c in (nchunk - 2, nchunk - 1):
            b = c % 2
            pltpu.make_async_copy(
                outs[b],
                out_hbm.at[pl.ds(c * chunk_rows, chunk_rows),
                           pl.ds(col0, cols_per_w)],
                osems[b]).wait()

        pltpu.async_copy(
            fix_v, out_hbm.at[clss_v, pl.ds(col0, cols_per_w)], fsem).wait()

    return sc_kernel


def kernel(x, alpha, beta, clss):
    R, C = x.shape
    K = clss.shape[0]
    assert R % (32 * L) == 0 and C % 5 == 0 and (C // 5) % 8 == 0

    ab = jnp.concatenate([
        jnp.broadcast_to(alpha.astype(jnp.float32), (L,)),
        jnp.broadcast_to(beta.astype(jnp.float32), (L,)),
    ])
    sc = _build_sc_kernel(C, R, K)
    out_t = sc(x.T, ab, clss.astype(jnp.int32))
    return out_t.T


# speedup vs baseline: 1.0314x; 1.0314x over previous
import functools

import jax
import jax.numpy as jnp
from jax import lax
from jax.experimental import pallas as pl
from jax.experimental.pallas import tpu as pltpu
from jax.experimental.pallas import tpu_sc as plsc

L = 16


def _build_sc_kernel(N, M, K):
    NW = 32
    cols_per_w = M // NW
    nchunk = 5
    chunk_rows = N // nchunk
    kv = cols_per_w // L
    K_pad = -(-K // L) * L

    mesh = plsc.VectorSubcoreMesh(core_axis_name="c", subcore_axis_name="s")

    @functools.partial(
        pl.kernel,
        mesh=mesh,
        compiler_params=pltpu.CompilerParams(needs_layout_passes=False),
        out_type=jax.ShapeDtypeStruct((N, M), jnp.float32),
        scratch_types=[
            pltpu.VMEM((2 * L,), jnp.float32),
            pltpu.VMEM((K_pad,), jnp.int32),
            pltpu.VMEM((N,), jnp.float32),
            pltpu.VMEM((N,), jnp.float32),
            pltpu.VMEM((chunk_rows, cols_per_w), jnp.float32),
            pltpu.VMEM((chunk_rows, cols_per_w), jnp.float32),
            pltpu.VMEM((chunk_rows, cols_per_w), jnp.float32),
            pltpu.VMEM((chunk_rows, cols_per_w), jnp.float32),
            pltpu.SemaphoreType.DMA,
            pltpu.SemaphoreType.DMA,
            pltpu.SemaphoreType.DMA,
            pltpu.SemaphoreType.DMA,
        ],
    )
    def sc_kernel(xt_hbm, ab_hbm, clss_hbm, out_hbm,
                  ab_v, clss_v, a_v, b_v, in0, in1, out0, out1,
                  isem0, isem1, osem0, osem1):
        wid = lax.axis_index("s") * 2 + lax.axis_index("c")
        col0 = wid * cols_per_w

        ins = [in0, in1]
        outs = [out0, out1]
        isems = [isem0, isem1]
        osems = [osem0, osem1]

        for c in range(2):
            pltpu.async_copy(
                xt_hbm.at[pl.ds(c * chunk_rows, chunk_rows),
                          pl.ds(col0, cols_per_w)],
                ins[c], isems[c])

        pltpu.sync_copy(ab_hbm, ab_v)
        pltpu.sync_copy(clss_hbm, clss_v.at[pl.ds(0, K)])

        ones = jnp.full((L,), 1.0, jnp.float32)

        @plsc.parallel_loop(0, N // L)
        def init_body(i):
            a_v[pl.ds(i * L, L)] = ones
            b_v[pl.ds(i * L, L)] = ones

        zero16 = jnp.zeros((L,), jnp.int32)
        alpha_vec = ab_v[pl.ds(0, L)]
        beta_vec = ab_v[pl.ds(L, L)]
        lane = lax.iota(jnp.int32, L)
        for k in range(K_pad // L):
            idx = clss_v[pl.ds(k * L, L)]
            mask = (lane + (k * L)) < K
            plsc.store_scatter(a_v, [idx], alpha_vec, mask=mask)
            plsc.store_scatter(b_v, [idx], beta_vec, mask=mask)

        def do_chunk(c, b):
            row0 = pl.multiple_of(c * chunk_rows, 8)
            pltpu.make_async_copy(
                xt_hbm.at[pl.ds(row0, chunk_rows),
                          pl.ds(col0, cols_per_w)],
                ins[b], isems[b]).wait()

            @pl.when(c >= 2)
            def _wait_out(b=b, c=c):
                prev0 = pl.multiple_of((c - 2) * chunk_rows, 8)
                pltpu.make_async_copy(
                    outs[b],
                    out_hbm.at[pl.ds(prev0, chunk_rows),
                               pl.ds(col0, cols_per_w)],
                    osems[b]).wait()

            @plsc.parallel_loop(0, chunk_rows)
            def row_body(i, b=b, row0=row0):
                jv = zero16 + (row0 + i)
                av = plsc.load_gather(a_v, [jv])
                bv = plsc.load_gather(b_v, [jv])
                for k in range(kv):
                    outs[b][i, pl.ds(k * L, L)] = (
                        av * ins[b][i, pl.ds(k * L, L)] + bv)

            pltpu.async_copy(
                outs[b],
                out_hbm.at[pl.ds(row0, chunk_rows),
                           pl.ds(col0, cols_per_w)],
                osems[b])

            @pl.when(c + 2 < nchunk)
            def _start_next(b=b, c=c):
                nxt0 = pl.multiple_of((c + 2) * chunk_rows, 8)
                pltpu.async_copy(
                    xt_hbm.at[pl.ds(nxt0, chunk_rows),
                              pl.ds(col0, cols_per_w)],
                    ins[b], isems[b])

        @pl.loop(0, nchunk - 1, step=2)
        def group_body(g):
            for b in (0, 1):
                do_chunk(g + b, b)

        do_chunk(nchunk - 1, (nchunk - 1) % 2)

        for c in (nchunk - 2, nchunk - 1):
            b = c % 2
            pltpu.make_async_copy(
                outs[b],
                out_hbm.at[pl.ds(c * chunk_rows, chunk_rows),
                           pl.ds(col0, cols_per_w)],
                osems[b]).wait()

    return sc_kernel


def kernel(x, alpha, beta, clss):
    R, C = x.shape
    K = clss.shape[0]
    assert R % (32 * L) == 0 and C % 5 == 0 and (C // 5) % 8 == 0

    ab = jnp.concatenate([
        jnp.broadcast_to(alpha.astype(jnp.float32), (L,)),
        jnp.broadcast_to(beta.astype(jnp.float32), (L,)),
    ])
    sc = _build_sc_kernel(C, R, K)
    out_t = sc(x.T, ab, clss.astype(jnp.int32))
    return out_t.T
